# baseline (speedup 1.0000x reference)
import functools

import jax
import jax.numpy as jnp
from jax import lax
from jax.experimental import pallas as pl
from jax.experimental.pallas import tpu as pltpu, tpu_sc as plsc

_NSAMPLE = 32
_QBLK = 8


def _topk_body(q_ref, x_ref, idx_ref):
    b = pl.program_id(0)
    n = x_ref.shape[2]
    qblk = q_ref.shape[1]

    x = x_ref[0]
    q = q_ref[0]

    d = None
    for c in range(3):
        diff = q[:, c : c + 1] - x[c : c + 1, :]
        sq = diff * diff
        d = sq if d is None else d + sq

    iota_n = lax.broadcasted_iota(jnp.int32, (qblk, n), 1)
    cols = []
    for _ in range(_NSAMPLE):
        m = jnp.min(d, axis=1, keepdims=True)
        idx = jnp.min(jnp.where(d == m, iota_n, n), axis=1, keepdims=True)
        cols.append(idx)
        d = jnp.where(iota_n == idx, jnp.inf, d)

    idx_ref[0] = jnp.concatenate(cols, axis=1) + b * n


def _topk_indices(new_xyz, xyz_t):
    b, p, _ = new_xyz.shape
    n = xyz_t.shape[2]
    grid = (b, p // _QBLK)
    return pl.pallas_call(
        _topk_body,
        grid=grid,
        in_specs=[
            pl.BlockSpec((1, _QBLK, 3), lambda i, j: (i, j, 0)),
            pl.BlockSpec((1, 3, n), lambda i, j: (i, 0, 0)),
        ],
        out_specs=pl.BlockSpec((1, _QBLK, _NSAMPLE), lambda i, j: (i, j, 0)),
        out_shape=jax.ShapeDtypeStruct((b, p, _NSAMPLE), jnp.int32),
    )(new_xyz, xyz_t)


_GCHUNK = 128


def _gather_rows(table, idx_flat):
    m, d = idx_flat.shape[0], table.shape[1]
    mesh = plsc.VectorSubcoreMesh(core_axis_name="c", subcore_axis_name="s")
    nw = mesh.num_cores * mesh.num_subcores
    per_w = m // nw
    n_chunks = per_w // _GCHUNK

    @functools.partial(
        pl.kernel,
        mesh=mesh,
        out_type=jax.ShapeDtypeStruct((m, d), jnp.float32),
        scratch_types=[
            pltpu.VMEM((_GCHUNK,), jnp.int32),
            pltpu.VMEM((_GCHUNK, d), jnp.float32),
            pltpu.SemaphoreType.DMA,
        ],
        compiler_params=pltpu.CompilerParams(use_tc_tiling_on_sc=False),
    )
    def gather_kernel(table_hbm, idx_hbm, out_hbm, idx_v, rows_v, sem):
        wid = lax.axis_index("s") * mesh.num_cores + lax.axis_index("c")
        base = wid * per_w

        def chunk(i, carry):
            off = base + i * _GCHUNK
            pltpu.sync_copy(idx_hbm.at[pl.ds(off, _GCHUNK)], idx_v)
            pltpu.async_copy(table_hbm.at[idx_v], rows_v, sem).wait()
            pltpu.sync_copy(rows_v, out_hbm.at[pl.ds(off, _GCHUNK)])
            return carry

        lax.fori_loop(0, n_chunks, chunk, 0)

    return gather_kernel(table, idx_flat)


def kernel(xyz, new_xyz, components, new_components, features):
    del components, new_components
    b, n, _ = xyz.shape
    p = new_xyz.shape[1]
    c = features.shape[1]
    s = _NSAMPLE

    xyz_t = jnp.transpose(xyz, (0, 2, 1))
    idx = _topk_indices(new_xyz, xyz_t)

    d_pad = 80
    table = jnp.concatenate(
        [
            xyz,
            jnp.transpose(features, (0, 2, 1)),
            jnp.zeros((b, n, d_pad - 3 - c), jnp.float32),
        ],
        axis=-1,
    ).reshape(b * n, d_pad)

    rows = _gather_rows(table, idx.reshape(-1))
    rows = rows.reshape(b, p, s, d_pad)

    grouped_xyz = rows[..., :3] - new_xyz[:, :, None, :]
    grouped_feat = rows[..., 3 : 3 + c]
    out = jnp.concatenate([grouped_xyz, grouped_feat], axis=-1)
    return jnp.transpose(out, (0, 3, 1, 2))


# speedup vs baseline: 4.5842x; 4.5842x over previous
import functools

import jax
import jax.numpy as jnp
from jax import lax
from jax.experimental import pallas as pl
from jax.experimental.pallas import tpu as pltpu, tpu_sc as plsc

_NSAMPLE = 32
_QBLK = 256
_CHUNK = 128


def _topk_body(q_ref, x_ref, idx_ref, d_ref):
    b = pl.program_id(0)
    n = x_ref.shape[2]
    qblk = q_ref.shape[1]
    n_chunks = n // _CHUNK

    x = x_ref[0]
    q = q_ref[0]

    d = None
    for c in range(3):
        diff = q[:, c : c + 1] - x[c : c + 1, :]
        sq = diff * diff
        d = sq if d is None else d + sq
    d_ref[...] = d

    lane_iota = lax.broadcasted_iota(jnp.int32, (qblk, _CHUNK), 1)
    iota_s = lax.broadcasted_iota(jnp.int32, (qblk, _NSAMPLE), 1)

    def step(s, carry):
        prev_idx, acc = carry
        vm = None
        vi = None
        for k in range(n_chunks):
            sl = pl.ds(k * _CHUNK, _CHUNK)
            ik = lane_iota + (k * _CHUNK)
            dk = d_ref[:, sl]
            dk = jnp.where(ik == prev_idx, jnp.inf, dk)
            d_ref[:, sl] = dk
            if vm is None:
                vm, vi = dk, ik
            else:
                take = dk < vm
                vm = jnp.where(take, dk, vm)
                vi = jnp.where(take, ik, vi)
        m = jnp.min(vm, axis=1, keepdims=True)
        idx = jnp.min(jnp.where(vm == m, vi, n), axis=1, keepdims=True)
        acc = jnp.where(iota_s == s, idx, acc)
        return idx, acc

    _, acc = lax.fori_loop(
        0,
        _NSAMPLE,
        step,
        (jnp.full((qblk, 1), -1, jnp.int32), jnp.zeros((qblk, _NSAMPLE), jnp.int32)),
    )
    idx_ref[0] = acc + b * n


def _topk_indices(new_xyz, xyz_t):
    b, p, _ = new_xyz.shape
    n = xyz_t.shape[2]
    grid = (b, p // _QBLK)
    return pl.pallas_call(
        _topk_body,
        grid=grid,
        in_specs=[
            pl.BlockSpec((1, _QBLK, 3), lambda i, j: (i, j, 0)),
            pl.BlockSpec((1, 3, n), lambda i, j: (i, 0, 0)),
        ],
        out_specs=pl.BlockSpec((1, _QBLK, _NSAMPLE), lambda i, j: (i, j, 0)),
        out_shape=jax.ShapeDtypeStruct((b, p, _NSAMPLE), jnp.int32),
        scratch_shapes=[pltpu.VMEM((_QBLK, n), jnp.float32)],
    )(new_xyz, xyz_t)


_GCHUNK = 128


def _gather_rows(table, idx_flat):
    m, d = idx_flat.shape[0], table.shape[1]
    mesh = plsc.VectorSubcoreMesh(core_axis_name="c", subcore_axis_name="s")
    nw = mesh.num_cores * mesh.num_subcores
    per_w = m // nw
    n_chunks = per_w // _GCHUNK

    @functools.partial(
        pl.kernel,
        mesh=mesh,
        out_type=jax.ShapeDtypeStruct((m, d), jnp.float32),
        scratch_types=[
            pltpu.VMEM((_GCHUNK,), jnp.int32),
            pltpu.VMEM((_GCHUNK, d), jnp.float32),
            pltpu.SemaphoreType.DMA,
        ],
        compiler_params=pltpu.CompilerParams(use_tc_tiling_on_sc=False),
    )
    def gather_kernel(table_hbm, idx_hbm, out_hbm, idx_v, rows_v, sem):
        wid = lax.axis_index("s") * mesh.num_cores + lax.axis_index("c")
        base = wid * per_w

        def chunk(i, carry):
            off = base + i * _GCHUNK
            pltpu.sync_copy(idx_hbm.at[pl.ds(off, _GCHUNK)], idx_v)
            pltpu.async_copy(table_hbm.at[idx_v], rows_v, sem).wait()
            pltpu.sync_copy(rows_v, out_hbm.at[pl.ds(off, _GCHUNK)])
            return carry

        lax.fori_loop(0, n_chunks, chunk, 0)

    return gather_kernel(table, idx_flat)


def kernel(xyz, new_xyz, components, new_components, features):
    del components, new_components
    b, n, _ = xyz.shape
    p = new_xyz.shape[1]
    c = features.shape[1]
    s = _NSAMPLE

    xyz_t = jnp.transpose(xyz, (0, 2, 1))
    idx = _topk_indices(new_xyz, xyz_t)

    d_pad = 80
    table = jnp.concatenate(
        [
            xyz,
            jnp.transpose(features, (0, 2, 1)),
            jnp.zeros((b, n, d_pad - 3 - c), jnp.float32),
        ],
        axis=-1,
    ).reshape(b * n, d_pad)

    rows = _gather_rows(table, idx.reshape(-1))
    rows = rows.reshape(b, p, s, d_pad)

    grouped_xyz = rows[..., :3] - new_xyz[:, :, None, :]
    grouped_feat = rows[..., 3 : 3 + c]
    out = jnp.concatenate([grouped_xyz, grouped_feat], axis=-1)
    return jnp.transpose(out, (0, 3, 1, 2))
